# baseline (speedup 1.0000x reference)
import functools

import jax
import jax.numpy as jnp
from jax import lax
from jax.experimental import pallas as pl
from jax.experimental.pallas import tpu as pltpu
from jax.experimental.pallas import tpu_sc as plsc

_V = 100000
_D = 32
_B = 16384
_L = 50

_NW = 32
_PER_W = _B // _NW
_C = 16
_NCHUNK = _PER_W // _C
_FL = _C * _L
_FL_PIECES = [(p * 128, 128) for p in range(_FL // 128)]
if _FL % 128:
    _FL_PIECES.append((_FL - _FL % 128, _FL % 128))


def _cont_proj_body(x_ref, w_ref, b_ref, o_ref):
    o_ref[...] = (
        jnp.dot(x_ref[...], w_ref[...], preferred_element_type=jnp.float32)
        + b_ref[...]
    )


def _cont_proj(x, w, b):
    return pl.pallas_call(
        _cont_proj_body,
        out_shape=jax.ShapeDtypeStruct((_B, _D), jnp.float32),
    )(x, w, b.reshape(1, _D))


def _sc_body(o_idx, p_idx, v_idx, r_idx, f_idx, o_tab, p_tab, v_tab, r_tab,
             f_tab,
             out_o, out_p, out_v, out_r, out_f,
             idx_o, idx_p, idx_v, idx_r,
             idx_f0, idx_f1,
             sel_o0, sel_p0, sel_v0, sel_r0,
             sel_o1, sel_p1, sel_v1, sel_r1,
             rows_f0, rows_f1, fl0, fl1,
             sem_g0, sem_g1, sem_i0, sem_i1, sem_o0, sem_o1):
    wid = lax.axis_index("c") * 16 + lax.axis_index("s")
    w_base = wid * _PER_W

    idx_f = (idx_f0, idx_f1)
    sel = ((sel_o0, sel_p0, sel_v0, sel_r0),
           (sel_o1, sel_p1, sel_v1, sel_r1))
    rows_f = (rows_f0, rows_f1)
    fl = (fl0, fl1)
    sem_g = (sem_g0, sem_g1)
    sem_i = (sem_i0, sem_i1)
    sem_o = (sem_o0, sem_o1)
    tabs = (o_tab, p_tab, v_tab, r_tab)
    idxs = (idx_o, idx_p, idx_v, idx_r)
    outs = (out_o, out_p, out_v, out_r, out_f)

    def fire_idx(g, b):
        base = pl.multiple_of((w_base + g * _C) * _L, _FL)
        return pltpu.async_copy(f_idx.at[pl.ds(base, _FL)], idx_f[b], sem_i[b])

    def drain_idx(b):
        pltpu.make_async_copy(f_idx.at[pl.ds(0, _FL)], idx_f[b],
                              sem_i[b]).wait()

    def fire_gathers(g, b):
        off = pl.multiple_of(g * _C, _C)
        for tab, ix, dst in zip(tabs, idxs, sel[b]):
            pltpu.async_copy(tab.at[ix.at[pl.ds(off, _C)]], dst, sem_g[b])
        for start, n in _FL_PIECES:
            pltpu.async_copy(f_tab.at[idx_f[b].at[pl.ds(start, n)]],
                             rows_f[b].at[pl.ds(start, n)], sem_g[b])

    def drain_gathers(b):
        for tab, dst in zip(tabs, sel[b]):
            pltpu.make_async_copy(tab.at[pl.ds(0, _C)], dst, sem_g[b]).wait()
        pltpu.make_async_copy(f_tab.at[pl.ds(0, _FL)], rows_f[b],
                              sem_g[b]).wait()

    def pool(b):
        rf, dst = rows_f[b], fl[b]

        def body(i, c2):
            row0 = i * _L
            acc_lo = rf[row0, pl.ds(0, 16)]
            acc_hi = rf[row0, pl.ds(16, 16)]
            for j in range(1, _L):
                acc_lo = acc_lo + rf[row0 + j, pl.ds(0, 16)]
                acc_hi = acc_hi + rf[row0 + j, pl.ds(16, 16)]
            dst[i, pl.ds(0, 16)] = acc_lo * (1.0 / _L)
            dst[i, pl.ds(16, 16)] = acc_hi * (1.0 / _L)
            return c2

        lax.fori_loop(0, _C, body, 0)

    def fire_out(g, b):
        base = pl.multiple_of(w_base + g * _C, _C)
        for k, src in enumerate(sel[b] + (fl[b],)):
            pltpu.async_copy(src, outs[k].at[pl.ds(base, _C)], sem_o[b])

    def drain_out(b):
        for k, src in enumerate(sel[b] + (fl[b],)):
            pltpu.make_async_copy(src, outs[k].at[pl.ds(0, _C)],
                                  sem_o[b]).wait()

    pltpu.sync_copy(o_idx.at[pl.ds(w_base, _PER_W)], idx_o)
    pltpu.sync_copy(p_idx.at[pl.ds(w_base, _PER_W)], idx_p)
    pltpu.sync_copy(v_idx.at[pl.ds(w_base, _PER_W)], idx_v)
    pltpu.sync_copy(r_idx.at[pl.ds(w_base, _PER_W)], idx_r)

    fire_idx(0, 0)
    fire_idx(1, 1)
    drain_idx(0)
    fire_gathers(0, 0)

    drain_gathers(0)
    drain_idx(1)
    fire_gathers(1, 1)
    fire_idx(2, 0)
    pool(0)
    fire_out(0, 0)
    drain_gathers(1)
    drain_idx(0)
    drain_out(0)
    fire_gathers(2, 0)
    fire_idx(3, 1)
    pool(1)
    fire_out(1, 1)

    def pair(gg, carry):
        for b in (0, 1):
            g = 2 + gg * 2 + b
            nb = 1 - b
            drain_gathers(b)
            drain_idx(nb)
            drain_out(nb)
            fire_gathers(g + 1, nb)
            fire_idx(g + 2, b)
            pool(b)
            fire_out(g, b)
        return carry

    lax.fori_loop(0, (_NCHUNK - 4) // 2, pair, 0)

    drain_gathers(0)
    drain_idx(1)
    drain_out(1)
    fire_gathers(_NCHUNK - 1, 1)
    pool(0)
    fire_out(_NCHUNK - 2, 0)

    drain_gathers(1)
    drain_out(0)
    pool(1)
    fire_out(_NCHUNK - 1, 1)
    drain_out(1)


_sc_kernel = functools.partial(
    pl.kernel,
    mesh=plsc.VectorSubcoreMesh(core_axis_name="c", subcore_axis_name="s"),
    out_type=tuple(
        jax.ShapeDtypeStruct((_B, _D), jnp.float32) for _ in range(5)
    ),
    compiler_params=pltpu.CompilerParams(use_tc_tiling_on_sc=False),
    scratch_types=(
        [pltpu.VMEM((_PER_W,), jnp.int32)] * 4
        + [pltpu.VMEM((_FL,), jnp.int32)] * 2
        + [pltpu.VMEM((_C, _D), jnp.float32)] * 8
        + [pltpu.VMEM((_FL, _D), jnp.float32)] * 2
        + [pltpu.VMEM((_C, _D), jnp.float32)] * 2
        + [pltpu.SemaphoreType.DMA] * 6
    ),
)(_sc_body)


def kernel(origin_idx, process_idx, variety_idx, roast_idx, flavor_idx,
           continuous_features, origin_table, process_table, variety_table,
           roast_table, flavor_table, W_cont, b_cont):
    cont_emb = _cont_proj(continuous_features, W_cont, b_cont)
    f_idx_flat = flavor_idx.reshape(_B * _L)
    e_o, e_p, e_v, e_r, e_f = _sc_kernel(
        origin_idx, process_idx, variety_idx, roast_idx, f_idx_flat,
        origin_table, process_table, variety_table, roast_table, flavor_table,
    )
    return jnp.concatenate([e_o, e_p, e_v, e_r, e_f, cont_emb], axis=1)


# speedup vs baseline: 1.0264x; 1.0264x over previous
import functools

import jax
import jax.numpy as jnp
from jax import lax
from jax.experimental import pallas as pl
from jax.experimental.pallas import tpu as pltpu
from jax.experimental.pallas import tpu_sc as plsc

_V = 100000
_D = 32
_B = 16384
_L = 50

_NW = 32
_PER_W = _B // _NW
_C = 32
_NCHUNK = _PER_W // _C
_FL = _C * _L
_FL_PIECES = [(p * 128, 128) for p in range(_FL // 128)]
if _FL % 128:
    _FL_PIECES.append((_FL - _FL % 128, _FL % 128))


def _cont_proj_body(x_ref, w_ref, b_ref, o_ref):
    o_ref[...] = (
        jnp.dot(x_ref[...], w_ref[...], preferred_element_type=jnp.float32)
        + b_ref[...]
    )


def _cont_proj(x, w, b):
    return pl.pallas_call(
        _cont_proj_body,
        out_shape=jax.ShapeDtypeStruct((_B, _D), jnp.float32),
    )(x, w, b.reshape(1, _D))


def _sc_body(o_idx, p_idx, v_idx, r_idx, f_idx, o_tab, p_tab, v_tab, r_tab,
             f_tab,
             out_o, out_p, out_v, out_r, out_f,
             idx_o, idx_p, idx_v, idx_r,
             idx_f0, idx_f1,
             sel_o0, sel_p0, sel_v0, sel_r0,
             sel_o1, sel_p1, sel_v1, sel_r1,
             rows_f0, rows_f1, fl0, fl1,
             sem_g0, sem_g1, sem_i0, sem_i1, sem_o0, sem_o1):
    wid = lax.axis_index("c") * 16 + lax.axis_index("s")
    w_base = wid * _PER_W

    idx_f = (idx_f0, idx_f1)
    sel = ((sel_o0, sel_p0, sel_v0, sel_r0),
           (sel_o1, sel_p1, sel_v1, sel_r1))
    rows_f = (rows_f0, rows_f1)
    fl = (fl0, fl1)
    sem_g = (sem_g0, sem_g1)
    sem_i = (sem_i0, sem_i1)
    sem_o = (sem_o0, sem_o1)
    tabs = (o_tab, p_tab, v_tab, r_tab)
    idxs = (idx_o, idx_p, idx_v, idx_r)
    outs = (out_o, out_p, out_v, out_r, out_f)

    def fire_idx(g, b):
        base = pl.multiple_of((w_base + g * _C) * _L, _FL)
        return pltpu.async_copy(f_idx.at[pl.ds(base, _FL)], idx_f[b], sem_i[b])

    def drain_idx(b):
        pltpu.make_async_copy(f_idx.at[pl.ds(0, _FL)], idx_f[b],
                              sem_i[b]).wait()

    def fire_gathers(g, b):
        off = pl.multiple_of(g * _C, _C)
        for tab, ix, dst in zip(tabs, idxs, sel[b]):
            pltpu.async_copy(tab.at[ix.at[pl.ds(off, _C)]], dst, sem_g[b])
        for start, n in _FL_PIECES:
            pltpu.async_copy(f_tab.at[idx_f[b].at[pl.ds(start, n)]],
                             rows_f[b].at[pl.ds(start, n)], sem_g[b])

    def drain_gathers(b):
        for tab, dst in zip(tabs, sel[b]):
            pltpu.make_async_copy(tab.at[pl.ds(0, _C)], dst, sem_g[b]).wait()
        pltpu.make_async_copy(f_tab.at[pl.ds(0, _FL)], rows_f[b],
                              sem_g[b]).wait()

    def pool(b):
        rf, dst = rows_f[b], fl[b]

        def body(i, c2):
            row0 = i * _L
            acc_lo = rf[row0, pl.ds(0, 16)]
            acc_hi = rf[row0, pl.ds(16, 16)]
            for j in range(1, _L):
                acc_lo = acc_lo + rf[row0 + j, pl.ds(0, 16)]
                acc_hi = acc_hi + rf[row0 + j, pl.ds(16, 16)]
            dst[i, pl.ds(0, 16)] = acc_lo * (1.0 / _L)
            dst[i, pl.ds(16, 16)] = acc_hi * (1.0 / _L)
            return c2

        lax.fori_loop(0, _C, body, 0)

    def fire_out(g, b):
        base = pl.multiple_of(w_base + g * _C, _C)
        for k, src in enumerate(sel[b] + (fl[b],)):
            pltpu.async_copy(src, outs[k].at[pl.ds(base, _C)], sem_o[b])

    def drain_out(b):
        for k, src in enumerate(sel[b] + (fl[b],)):
            pltpu.make_async_copy(src, outs[k].at[pl.ds(0, _C)],
                                  sem_o[b]).wait()

    pltpu.sync_copy(o_idx.at[pl.ds(w_base, _PER_W)], idx_o)
    pltpu.sync_copy(p_idx.at[pl.ds(w_base, _PER_W)], idx_p)
    pltpu.sync_copy(v_idx.at[pl.ds(w_base, _PER_W)], idx_v)
    pltpu.sync_copy(r_idx.at[pl.ds(w_base, _PER_W)], idx_r)

    fire_idx(0, 0)
    fire_idx(1, 1)
    drain_idx(0)
    fire_gathers(0, 0)

    drain_gathers(0)
    drain_idx(1)
    fire_gathers(1, 1)
    fire_idx(2, 0)
    pool(0)
    fire_out(0, 0)
    drain_gathers(1)
    drain_idx(0)
    drain_out(0)
    fire_gathers(2, 0)
    fire_idx(3, 1)
    pool(1)
    fire_out(1, 1)

    def pair(gg, carry):
        for b in (0, 1):
            g = 2 + gg * 2 + b
            nb = 1 - b
            drain_gathers(b)
            drain_idx(nb)
            drain_out(nb)
            fire_gathers(g + 1, nb)
            fire_idx(g + 2, b)
            pool(b)
            fire_out(g, b)
        return carry

    lax.fori_loop(0, (_NCHUNK - 4) // 2, pair, 0)

    drain_gathers(0)
    drain_idx(1)
    drain_out(1)
    fire_gathers(_NCHUNK - 1, 1)
    pool(0)
    fire_out(_NCHUNK - 2, 0)

    drain_gathers(1)
    drain_out(0)
    pool(1)
    fire_out(_NCHUNK - 1, 1)
    drain_out(1)


_sc_kernel = functools.partial(
    pl.kernel,
    mesh=plsc.VectorSubcoreMesh(core_axis_name="c", subcore_axis_name="s"),
    out_type=tuple(
        jax.ShapeDtypeStruct((_B, _D), jnp.float32) for _ in range(5)
    ),
    compiler_params=pltpu.CompilerParams(use_tc_tiling_on_sc=False),
    scratch_types=(
        [pltpu.VMEM((_PER_W,), jnp.int32)] * 4
        + [pltpu.VMEM((_FL,), jnp.int32)] * 2
        + [pltpu.VMEM((_C, _D), jnp.float32)] * 8
        + [pltpu.VMEM((_FL, _D), jnp.float32)] * 2
        + [pltpu.VMEM((_C, _D), jnp.float32)] * 2
        + [pltpu.SemaphoreType.DMA] * 6
    ),
)(_sc_body)


def kernel(origin_idx, process_idx, variety_idx, roast_idx, flavor_idx,
           continuous_features, origin_table, process_table, variety_table,
           roast_table, flavor_table, W_cont, b_cont):
    cont_emb = _cont_proj(continuous_features, W_cont, b_cont)
    f_idx_flat = flavor_idx.reshape(_B * _L)
    e_o, e_p, e_v, e_r, e_f = _sc_kernel(
        origin_idx, process_idx, variety_idx, roast_idx, f_idx_flat,
        origin_table, process_table, variety_table, roast_table, flavor_table,
    )
    return jnp.concatenate([e_o, e_p, e_v, e_r, e_f, cont_emb], axis=1)


# speedup vs baseline: 1.0889x; 1.0609x over previous
import functools

import jax
import jax.numpy as jnp
from jax import lax
from jax.experimental import pallas as pl
from jax.experimental.pallas import tpu as pltpu
from jax.experimental.pallas import tpu_sc as plsc

_V = 100000
_D = 32
_B = 16384
_L = 50

_NW = 32
_PER_W = _B // _NW
_C = 32
_NCHUNK = _PER_W // _C
_FL = _C * _L
_FL_PIECES = [(p * 128, 128) for p in range(_FL // 128)]
if _FL % 128:
    _FL_PIECES.append((_FL - _FL % 128, _FL % 128))


def _cont_proj_body(x_ref, w_ref, b_ref, o_ref):
    o_ref[...] = (
        jnp.dot(x_ref[...], w_ref[...], preferred_element_type=jnp.float32)
        + b_ref[...]
    )


def _cont_proj(x, w, b):
    return pl.pallas_call(
        _cont_proj_body,
        out_shape=jax.ShapeDtypeStruct((_B, _D), jnp.float32),
    )(x, w, b.reshape(1, _D))


def _sc_body(o_idx, p_idx, v_idx, r_idx, f_idx, o_tab, p_tab, v_tab, r_tab,
             f_tab, cont,
             out,
             idx_o, idx_p, idx_v, idx_r,
             idx_f0, idx_f1,
             sel_o0, sel_p0, sel_v0, sel_r0,
             sel_o1, sel_p1, sel_v1, sel_r1,
             rows_f0, rows_f1, fl0, fl1,
             sem_g0, sem_g1, sem_i0, sem_i1, sem_o0, sem_o1, sem_c):
    wid = lax.axis_index("c") * 16 + lax.axis_index("s")
    w_base = wid * _PER_W

    idx_f = (idx_f0, idx_f1)
    sel = ((sel_o0, sel_p0, sel_v0, sel_r0),
           (sel_o1, sel_p1, sel_v1, sel_r1))
    rows_f = (rows_f0, rows_f1)
    fl = (fl0, fl1)
    sem_g = (sem_g0, sem_g1)
    sem_i = (sem_i0, sem_i1)
    sem_o = (sem_o0, sem_o1)
    tabs = (o_tab, p_tab, v_tab, r_tab)
    idxs = (idx_o, idx_p, idx_v, idx_r)

    def fire_idx(g, b):
        base = pl.multiple_of((w_base + g * _C) * _L, _FL)
        return pltpu.async_copy(f_idx.at[pl.ds(base, _FL)], idx_f[b], sem_i[b])

    def drain_idx(b):
        pltpu.make_async_copy(f_idx.at[pl.ds(0, _FL)], idx_f[b],
                              sem_i[b]).wait()

    def fire_gathers(g, b):
        off = pl.multiple_of(g * _C, _C)
        for tab, ix, dst in zip(tabs, idxs, sel[b]):
            pltpu.async_copy(tab.at[ix.at[pl.ds(off, _C)]], dst, sem_g[b])
        for start, n in _FL_PIECES:
            pltpu.async_copy(f_tab.at[idx_f[b].at[pl.ds(start, n)]],
                             rows_f[b].at[pl.ds(start, n)], sem_g[b])

    def drain_gathers(b):
        for tab, dst in zip(tabs, sel[b]):
            pltpu.make_async_copy(tab.at[pl.ds(0, _C)], dst, sem_g[b]).wait()
        pltpu.make_async_copy(f_tab.at[pl.ds(0, _FL)], rows_f[b],
                              sem_g[b]).wait()

    def pool(b):
        rf, dst = rows_f[b], fl[b]

        def body(i, c2):
            row0 = i * _L
            acc_lo = rf[row0, pl.ds(0, 16)]
            acc_hi = rf[row0, pl.ds(16, 16)]
            for j in range(1, _L):
                acc_lo = acc_lo + rf[row0 + j, pl.ds(0, 16)]
                acc_hi = acc_hi + rf[row0 + j, pl.ds(16, 16)]
            dst[i, pl.ds(0, 16)] = acc_lo * (1.0 / _L)
            dst[i, pl.ds(16, 16)] = acc_hi * (1.0 / _L)
            return c2

        lax.fori_loop(0, _C, body, 0)

    def fire_out(g, b):
        base = pl.multiple_of(w_base + g * _C, _C)
        for k, src in enumerate(sel[b] + (fl[b],)):
            pltpu.async_copy(
                src, out.at[pl.ds(base, _C), pl.ds(k * _D, _D)], sem_o[b]
            )

    def drain_out(b):
        for k, src in enumerate(sel[b] + (fl[b],)):
            pltpu.make_async_copy(
                src, out.at[pl.ds(0, _C), pl.ds(k * _D, _D)], sem_o[b]
            ).wait()

    pltpu.async_copy(
        cont.at[pl.ds(w_base, _PER_W)],
        out.at[pl.ds(w_base, _PER_W), pl.ds(5 * _D, _D)],
        sem_c,
    )

    pltpu.sync_copy(o_idx.at[pl.ds(w_base, _PER_W)], idx_o)
    pltpu.sync_copy(p_idx.at[pl.ds(w_base, _PER_W)], idx_p)
    pltpu.sync_copy(v_idx.at[pl.ds(w_base, _PER_W)], idx_v)
    pltpu.sync_copy(r_idx.at[pl.ds(w_base, _PER_W)], idx_r)

    fire_idx(0, 0)
    fire_idx(1, 1)
    drain_idx(0)
    fire_gathers(0, 0)

    drain_gathers(0)
    drain_idx(1)
    fire_gathers(1, 1)
    fire_idx(2, 0)
    pool(0)
    fire_out(0, 0)
    drain_gathers(1)
    drain_idx(0)
    drain_out(0)
    fire_gathers(2, 0)
    fire_idx(3, 1)
    pool(1)
    fire_out(1, 1)

    def pair(gg, carry):
        for b in (0, 1):
            g = 2 + gg * 2 + b
            nb = 1 - b
            drain_gathers(b)
            drain_idx(nb)
            drain_out(nb)
            fire_gathers(g + 1, nb)
            fire_idx(g + 2, b)
            pool(b)
            fire_out(g, b)
        return carry

    lax.fori_loop(0, (_NCHUNK - 4) // 2, pair, 0)

    drain_gathers(0)
    drain_idx(1)
    drain_out(1)
    fire_gathers(_NCHUNK - 1, 1)
    pool(0)
    fire_out(_NCHUNK - 2, 0)

    drain_gathers(1)
    drain_out(0)
    pool(1)
    fire_out(_NCHUNK - 1, 1)
    drain_out(1)
    pltpu.make_async_copy(
        cont.at[pl.ds(0, _PER_W)],
        out.at[pl.ds(0, _PER_W), pl.ds(5 * _D, _D)],
        sem_c,
    ).wait()


_sc_kernel = functools.partial(
    pl.kernel,
    mesh=plsc.VectorSubcoreMesh(core_axis_name="c", subcore_axis_name="s"),
    out_type=jax.ShapeDtypeStruct((_B, 6 * _D), jnp.float32),
    compiler_params=pltpu.CompilerParams(use_tc_tiling_on_sc=False),
    scratch_types=(
        [pltpu.VMEM((_PER_W,), jnp.int32)] * 4
        + [pltpu.VMEM((_FL,), jnp.int32)] * 2
        + [pltpu.VMEM((_C, _D), jnp.float32)] * 8
        + [pltpu.VMEM((_FL, _D), jnp.float32)] * 2
        + [pltpu.VMEM((_C, _D), jnp.float32)] * 2
        + [pltpu.SemaphoreType.DMA] * 7
    ),
)(_sc_body)


def kernel(origin_idx, process_idx, variety_idx, roast_idx, flavor_idx,
           continuous_features, origin_table, process_table, variety_table,
           roast_table, flavor_table, W_cont, b_cont):
    cont_emb = _cont_proj(continuous_features, W_cont, b_cont)
    f_idx_flat = flavor_idx.reshape(_B * _L)
    return _sc_kernel(
        origin_idx, process_idx, variety_idx, roast_idx, f_idx_flat,
        origin_table, process_table, variety_table, roast_table, flavor_table,
        cont_emb,
    )
